# baseline (speedup 1.0000x reference)
import functools

import jax
import jax.numpy as jnp
from jax import lax
from jax.experimental import pallas as pl
from jax.experimental.pallas import tpu as pltpu
from jax.experimental.pallas import tpu_sc as plsc

N = 10000
E = 320000
B = 128

NC = 2
NS = 16
NW = NC * NS
EPW = E // NW
CH = 128
NFULL = EPW // CH
REM = EPW - NFULL * CH
DEGW = 16
DRC = 80
NRC = N // DRC
MAXRC = (NRC + NS - 1) // NS

BN = 1000
NB = N // BN

_f32 = jnp.float32


@functools.lru_cache(maxsize=None)
def _sc_degree():
  dp = DEGW
  mesh = plsc.VectorSubcoreMesh(core_axis_name="c", subcore_axis_name="s")

  @functools.partial(
      pl.kernel,
      out_type=jax.ShapeDtypeStruct((NC, N, dp), _f32),
      mesh=mesh,
      scratch_types=[
          pltpu.VMEM((2, CH), jnp.int32),
          pltpu.VMEM((2, CH), jnp.int32),
          pltpu.VMEM((2, REM), jnp.int32),
          pltpu.VMEM((CH, dp), _f32),
          pltpu.SemaphoreType.DMA,
          pltpu.SemaphoreType.DMA,
          pltpu.VMEM_SHARED((N, dp), _f32),
      ],
      compiler_params=pltpu.CompilerParams(use_tc_tiling_on_sc=False),
  )
  def k(ew_hbm, ones_hbm, zeros_hbm, out_hbm,
        ib0, ib1, ibr, ones_v, si0, si1, acc_sh):
    cid = lax.axis_index("c")
    sid = lax.axis_index("s")
    wid = sid * NC + cid

    def idx_start(i, ib, sem):
      off = pl.multiple_of(i * CH, CH)
      pltpu.async_copy(ew_hbm.at[wid, :, pl.ds(off, CH)], ib, sem)

    def idx_wait(i, ib, sem):
      off = pl.multiple_of(i * CH, CH)
      pltpu.make_async_copy(ew_hbm.at[wid, :, pl.ds(off, CH)], ib, sem).wait()

    idx_start(0, ib0, si0)
    idx_start(1, ib1, si1)
    pltpu.sync_copy(ones_hbm, ones_v)

    def zbody(k, carry):
      c = sid + k * NS

      @pl.when(c < NRC)
      def _():
        r0 = pl.multiple_of(c * DRC, DRC)
        pltpu.sync_copy(zeros_hbm, acc_sh.at[pl.ds(r0, DRC)])

      return carry

    lax.fori_loop(0, MAXRC, zbody, 0)
    plsc.subcore_barrier()

    def ebody(j, carry):
      i0 = 2 * j
      idx_wait(i0, ib0, si0)
      pltpu.sync_copy(ones_v, acc_sh.at[ib0.at[1]], add=True)
      idx_start(i0 + 2, ib0, si0)
      idx_wait(i0 + 1, ib1, si1)
      pltpu.sync_copy(ones_v, acc_sh.at[ib1.at[1]], add=True)
      idx_start(jnp.minimum(i0 + 3, NFULL - 1), ib1, si1)
      return carry

    lax.fori_loop(0, (NFULL - 2) // 2, ebody, 0)
    idx_wait(NFULL - 2, ib0, si0)
    pltpu.sync_copy(ones_v, acc_sh.at[ib0.at[1]], add=True)
    idx_wait(NFULL - 1, ib1, si1)
    pltpu.sync_copy(ones_v, acc_sh.at[ib1.at[1]], add=True)
    pltpu.sync_copy(ew_hbm.at[wid, :, pl.ds(NFULL * CH, REM)], ibr)
    pltpu.sync_copy(ones_v.at[pl.ds(0, REM)], acc_sh.at[ibr.at[1]], add=True)
    plsc.subcore_barrier()

    def dbody(k, carry):
      c = sid + k * NS

      @pl.when(c < NRC)
      def _():
        r0 = pl.multiple_of(c * DRC, DRC)
        pltpu.sync_copy(acc_sh.at[pl.ds(r0, DRC)], out_hbm.at[cid, pl.ds(r0, DRC)])

      return carry

    lax.fori_loop(0, MAXRC, dbody, 0)

  return k


@functools.lru_cache(maxsize=None)
def _sc_scatter(dp: int):
  mesh = plsc.VectorSubcoreMesh(core_axis_name="c", subcore_axis_name="s")

  @functools.partial(
      pl.kernel,
      out_type=jax.ShapeDtypeStruct((NC, N, dp), _f32),
      mesh=mesh,
      scratch_types=[
          pltpu.VMEM((2, CH), jnp.int32),
          pltpu.VMEM((2, CH), jnp.int32),
          pltpu.VMEM((2, REM), jnp.int32),
          pltpu.VMEM((CH, dp), _f32),
          pltpu.VMEM((CH, dp), _f32),
          pltpu.VMEM((REM, dp), _f32),
          pltpu.SemaphoreType.DMA,
          pltpu.SemaphoreType.DMA,
          pltpu.SemaphoreType.DMA,
          pltpu.SemaphoreType.DMA,
          pltpu.VMEM_SHARED((N, dp), _f32),
      ],
      compiler_params=pltpu.CompilerParams(use_tc_tiling_on_sc=False),
  )
  def k(hs_hbm, ew_hbm, zeros_hbm, out_hbm,
        ib0, ib1, ibr, rows0, rows1, rowsr, si0, si1, sg0, sg1, acc_sh):
    cid = lax.axis_index("c")
    sid = lax.axis_index("s")
    wid = sid * NC + cid

    def idx_start(i, ib, sem):
      off = pl.multiple_of(i * CH, CH)
      pltpu.async_copy(ew_hbm.at[wid, :, pl.ds(off, CH)], ib, sem)

    def idx_wait(i, ib, sem):
      off = pl.multiple_of(i * CH, CH)
      pltpu.make_async_copy(ew_hbm.at[wid, :, pl.ds(off, CH)], ib, sem).wait()

    idx_start(0, ib0, si0)
    idx_start(1, ib1, si1)

    def zbody(k, carry):
      c = sid + k * NS

      @pl.when(c < NRC)
      def _():
        r0 = pl.multiple_of(c * DRC, DRC)
        pltpu.sync_copy(zeros_hbm, acc_sh.at[pl.ds(r0, DRC)])

      return carry

    lax.fori_loop(0, MAXRC, zbody, 0)
    plsc.subcore_barrier()

    idx_wait(0, ib0, si0)
    pltpu.async_copy(hs_hbm.at[ib0.at[0]], rows0, sg0)

    def ebody(j, carry):
      i0 = 2 * j
      pltpu.make_async_copy(hs_hbm.at[ib0.at[0]], rows0, sg0).wait()
      idx_wait(i0 + 1, ib1, si1)
      pltpu.async_copy(hs_hbm.at[ib1.at[0]], rows1, sg1)
      pltpu.sync_copy(rows0, acc_sh.at[ib0.at[1]], add=True)
      idx_start(i0 + 2, ib0, si0)
      pltpu.make_async_copy(hs_hbm.at[ib1.at[0]], rows1, sg1).wait()
      idx_wait(i0 + 2, ib0, si0)
      pltpu.async_copy(hs_hbm.at[ib0.at[0]], rows0, sg0)
      pltpu.sync_copy(rows1, acc_sh.at[ib1.at[1]], add=True)
      idx_start(jnp.minimum(i0 + 3, NFULL - 1), ib1, si1)
      return carry

    lax.fori_loop(0, (NFULL - 2) // 2, ebody, 0)
    pltpu.make_async_copy(hs_hbm.at[ib0.at[0]], rows0, sg0).wait()
    idx_wait(NFULL - 1, ib1, si1)
    pltpu.async_copy(hs_hbm.at[ib1.at[0]], rows1, sg1)
    pltpu.sync_copy(rows0, acc_sh.at[ib0.at[1]], add=True)
    pltpu.sync_copy(ew_hbm.at[wid, :, pl.ds(NFULL * CH, REM)], ibr)
    pltpu.make_async_copy(hs_hbm.at[ib1.at[0]], rows1, sg1).wait()
    pltpu.async_copy(hs_hbm.at[ibr.at[0]], rowsr, sg0)
    pltpu.sync_copy(rows1, acc_sh.at[ib1.at[1]], add=True)
    pltpu.make_async_copy(hs_hbm.at[ibr.at[0]], rowsr, sg0).wait()
    pltpu.sync_copy(rowsr, acc_sh.at[ibr.at[1]], add=True)
    plsc.subcore_barrier()

    def dbody(k, carry):
      c = sid + k * NS

      @pl.when(c < NRC)
      def _():
        r0 = pl.multiple_of(c * DRC, DRC)
        pltpu.sync_copy(acc_sh.at[pl.ds(r0, DRC)], out_hbm.at[cid, pl.ds(r0, DRC)])

      return carry

    lax.fori_loop(0, MAXRC, dbody, 0)

  return k


def _prep_body(x_ref, w_ref, degp_ref, ts_ref, dinv_ref):
  deg = degp_ref[0][:, 0:1] + degp_ref[1][:, 0:1] + 1.0
  dinv = deg ** -0.5
  dinv_ref[...] = dinv
  t = jnp.dot(x_ref[...], w_ref[...], preferred_element_type=_f32)
  ts_ref[...] = t * dinv


def _prep_call(x, w0p, degp):
  return pl.pallas_call(
      _prep_body,
      grid=(NB,),
      in_specs=[
          pl.BlockSpec((BN, 128), lambda i: (i, 0)),
          pl.BlockSpec((128, 112), lambda i: (0, 0)),
          pl.BlockSpec((2, BN, 16), lambda i: (0, i, 0)),
      ],
      out_specs=[
          pl.BlockSpec((BN, 112), lambda i: (i, 0)),
          pl.BlockSpec((BN, 1), lambda i: (i, 0)),
      ],
      out_shape=[
          jax.ShapeDtypeStruct((N, 112), _f32),
          jax.ShapeDtypeStruct((N, 1), _f32),
      ],
  )(x, w0p, degp)


def _comb_body(acc_ref, ts_ref, dinv_ref, b_ref, out_ref):
  dinv = dinv_ref[...]
  h = jnp.maximum((acc_ref[0] + acc_ref[1] + ts_ref[...]) * dinv + b_ref[...],
                  0.0)
  out_ref[...] = h * dinv


def _comb_call(acc, ts, dinv, bp):
  d = ts.shape[1]
  return pl.pallas_call(
      _comb_body,
      grid=(NB,),
      in_specs=[
          pl.BlockSpec((2, BN, d), lambda i: (0, i, 0)),
          pl.BlockSpec((BN, d), lambda i: (i, 0)),
          pl.BlockSpec((BN, 1), lambda i: (i, 0)),
          pl.BlockSpec((1, d), lambda i: (0, 0)),
      ],
      out_specs=pl.BlockSpec((BN, d), lambda i: (i, 0)),
      out_shape=jax.ShapeDtypeStruct((N, d), _f32),
  )(acc, ts, dinv, bp)


def _mm2_body(acc_ref, ts_ref, dinv_ref, w1_ref, b1_ref, w2_ref, out_ref):
  dinv = dinv_ref[...]
  p = (acc_ref[0] + acc_ref[1] + ts_ref[...]) * dinv
  h = jnp.maximum(
      jnp.dot(p, w1_ref[...], preferred_element_type=_f32) + b1_ref[...], 0.0)
  t = jnp.dot(h, w2_ref[...], preferred_element_type=_f32)
  out_ref[...] = t * dinv


def _mm2_call(acc, ts, dinv, w1p, b1, w2p):
  din = ts.shape[1]
  dmid = w1p.shape[1]
  dout = w2p.shape[1]
  return pl.pallas_call(
      _mm2_body,
      grid=(NB,),
      in_specs=[
          pl.BlockSpec((2, BN, din), lambda i: (0, i, 0)),
          pl.BlockSpec((BN, din), lambda i: (i, 0)),
          pl.BlockSpec((BN, 1), lambda i: (i, 0)),
          pl.BlockSpec((din, dmid), lambda i: (0, 0)),
          pl.BlockSpec((1, dmid), lambda i: (0, 0)),
          pl.BlockSpec((dmid, dout), lambda i: (0, 0)),
      ],
      out_specs=pl.BlockSpec((BN, dout), lambda i: (i, 0)),
      out_shape=jax.ShapeDtypeStruct((N, dout), _f32),
  )(acc, ts, dinv, w1p, b1, w2p)


def _mm1_body(acc_ref, ts_ref, dinv_ref, b_ref, w_ref, out_ref):
  dinv = dinv_ref[...]
  h = jnp.maximum((acc_ref[0] + acc_ref[1] + ts_ref[...]) * dinv + b_ref[...],
                  0.0)
  t = jnp.dot(h, w_ref[...], preferred_element_type=_f32)
  out_ref[...] = t * dinv


def _mm1_call(acc, ts, dinv, bp, wp):
  din = ts.shape[1]
  dout = wp.shape[1]
  return pl.pallas_call(
      _mm1_body,
      grid=(NB,),
      in_specs=[
          pl.BlockSpec((2, BN, din), lambda i: (0, i, 0)),
          pl.BlockSpec((BN, din), lambda i: (i, 0)),
          pl.BlockSpec((BN, 1), lambda i: (i, 0)),
          pl.BlockSpec((1, din), lambda i: (0, 0)),
          pl.BlockSpec((din, dout), lambda i: (0, 0)),
      ],
      out_specs=pl.BlockSpec((BN, dout), lambda i: (i, 0)),
      out_shape=jax.ShapeDtypeStruct((N, dout), _f32),
  )(acc, ts, dinv, bp, wp)


def _final_body(acc_ref, ts_ref, dinv_ref, bm_ref, b3_ref,
                wd0_ref, bd0_ref, wd1_ref, bd1_ref, wd2_ref, bd2_ref,
                wd3_ref, bd3_ref, wn_ref, bnn_ref,
                t_ref, mfa_ref, mfw_ref,
                outa_ref, outw_ref, g_ref):
  i = pl.program_id(0)
  dinv = dinv_ref[...]
  h4 = jnp.maximum((acc_ref[0] + acc_ref[1] + ts_ref[...]) * dinv + b3_ref[...],
                   0.0)
  onehot = (bm_ref[...] == lax.broadcasted_iota(jnp.int32, (BN, B), 1)
            ).astype(_f32)
  contrib = lax.dot_general(onehot, h4, (((0,), (0,)), ((), ())),
                            preferred_element_type=_f32)

  @pl.when(i == 0)
  def _():
    g_ref[...] = contrib

  @pl.when(i > 0)
  def _():
    g_ref[...] = g_ref[...] + contrib

  @pl.when(i == pl.num_programs(0) - 1)
  def _():
    g = jnp.maximum(g_ref[...], 0.0)
    g = jnp.maximum(
        jnp.dot(g, wd0_ref[...], preferred_element_type=_f32) + bd0_ref[...],
        0.0)
    g = jnp.maximum(
        jnp.dot(g, wd1_ref[...], preferred_element_type=_f32) + bd1_ref[...],
        0.0)
    g = jnp.maximum(
        jnp.dot(g, wd2_ref[...], preferred_element_type=_f32) + bd2_ref[...],
        0.0)
    g = jnp.maximum(
        jnp.dot(g, wd3_ref[...], preferred_element_type=_f32) + bd3_ref[...],
        0.0)
    coef = jnp.dot(g, wn_ref[...], preferred_element_type=_f32) + bnn_ref[...]
    alpha = coef[:, 0:1]
    b12 = coef[:, 1:2]
    b21 = coef[:, 2:3]
    sig = 1.0 / (1.0 + jnp.exp(-alpha))
    alpha = 0.2 * (1.0 + sig / 10.0 * (0.47 / 0.2))
    r_t = 62.36367 * (t_ref[...] + 273.15)
    tau12 = b12 / r_t
    tau21 = b21 / r_t
    g12 = jnp.exp(-tau12 * alpha)
    g21 = jnp.exp(-tau21 * alpha)
    mfa = mfa_ref[...]
    mfw = mfw_ref[...]
    den12 = mfw + mfa * g12
    den21 = mfa + mfw * g21
    q12 = g12 / den12
    q21 = g21 / den21
    outa_ref[...] = mfw * mfw * (tau21 * q21 * q21 + tau12 * g12 / (den12 * den12))
    outw_ref[...] = mfa * mfa * (tau12 * q12 * q12 + tau21 * g21 / (den21 * den21))


def _final_call(acc, ts, dinv, bm2, b3p, wd0p, bd0, wd1, bd1, wd2, bd2,
                wd3, bd3, wn, bnn, temp, mfa, mfw):
  full = lambda shape: pl.BlockSpec(shape, lambda i: tuple(0 for _ in shape))
  return pl.pallas_call(
      _final_body,
      grid=(NB,),
      in_specs=[
          pl.BlockSpec((2, BN, 144), lambda i: (0, i, 0)),
          pl.BlockSpec((BN, 144), lambda i: (i, 0)),
          pl.BlockSpec((BN, 1), lambda i: (i, 0)),
          pl.BlockSpec((BN, 1), lambda i: (i, 0)),
          full((1, 144)),
          full((144, 260)), full((1, 260)),
          full((260, 60)), full((1, 60)),
          full((60, 180)), full((1, 180)),
          full((180, 100)), full((1, 100)),
          full((100, 3)), full((1, 3)),
          full((B, 1)), full((B, 1)), full((B, 1)),
      ],
      out_specs=[
          pl.BlockSpec((B, 1), lambda i: (0, 0)),
          pl.BlockSpec((B, 1), lambda i: (0, 0)),
      ],
      out_shape=[
          jax.ShapeDtypeStruct((B, 1), _f32),
          jax.ShapeDtypeStruct((B, 1), _f32),
      ],
      scratch_shapes=[pltpu.VMEM((B, 144), _f32)],
  )(acc, ts, dinv, bm2, b3p, wd0p, bd0, wd1, bd1, wd2, bd2, wd3, bd3,
    wn, bnn, temp, mfa, mfw)


def kernel(x, edge_indices, batch_mapping, temperature, mean, std,
           mole_frac_amine, mole_frac_water,
           Wg0, bg0, Wg1, bg1, Wg2, bg2, Wg3, bg3,
           Wd0, bd0, Wd1, bd1, Wd2, bd2, Wd3, bd3, Wn, bn):
  ew = edge_indices.reshape(2, NW, EPW).transpose(1, 0, 2)

  ones16 = jnp.ones((CH, DEGW), _f32)
  z16 = jnp.zeros((DRC, DEGW), _f32)
  z112 = jnp.zeros((DRC, 112), _f32)
  z144 = jnp.zeros((DRC, 144), _f32)

  degp = _sc_degree()(ew, ones16, z16)

  w0p = jnp.pad(Wg0, ((0, 0), (0, 12)))
  ts0, dinv = _prep_call(x, w0p, degp)
  acc0 = _sc_scatter(112)(ts0, ew, z112)

  b0p = jnp.pad(bg0, (0, 12)).reshape(1, 112)
  ts1 = _comb_call(acc0, ts0, dinv, b0p)
  acc1 = _sc_scatter(112)(ts1, ew, z112)

  w1p = jnp.pad(Wg1, ((0, 12), (0, 0)))
  w2p = jnp.pad(Wg2, ((0, 0), (0, 4)))
  ts2 = _mm2_call(acc1, ts1, dinv, w1p, bg1.reshape(1, 420), w2p)
  acc2 = _sc_scatter(144)(ts2, ew, z144)

  b2p = jnp.pad(bg2, (0, 4)).reshape(1, 144)
  w3p = jnp.pad(Wg3, ((0, 4), (0, 4)))
  ts3 = _mm1_call(acc2, ts2, dinv, b2p, w3p)
  acc3 = _sc_scatter(144)(ts3, ew, z144)

  b3p = jnp.pad(bg3, (0, 4)).reshape(1, 144)
  wd0p = jnp.pad(Wd0, ((0, 4), (0, 0)))
  ln_a, ln_w = _final_call(
      acc3, ts3, dinv, batch_mapping.reshape(N, 1).astype(jnp.int32), b3p,
      wd0p, bd0.reshape(1, -1), Wd1, bd1.reshape(1, -1),
      Wd2, bd2.reshape(1, -1), Wd3, bd3.reshape(1, -1),
      Wn, bn.reshape(1, -1), temperature.reshape(B, 1),
      mole_frac_amine.reshape(B, 1), mole_frac_water.reshape(B, 1))
  return ln_a.reshape(B), ln_w.reshape(B)


# speedup vs baseline: 1.0710x; 1.0710x over previous
import functools

import jax
import jax.numpy as jnp
from jax import lax
from jax.experimental import pallas as pl
from jax.experimental.pallas import tpu as pltpu
from jax.experimental.pallas import tpu_sc as plsc

N = 10000
E = 320000
B = 128

NC = 2
NS = 16
NW = NC * NS
EPW = E // NW
CH = 128
NFULL = EPW // CH
REM = EPW - NFULL * CH
DEGW = 16
DRC = 80
NRC = N // DRC
MAXRC = (NRC + NS - 1) // NS

BN = 1000
NB = N // BN

_f32 = jnp.float32


@functools.lru_cache(maxsize=None)
def _sc_degree():
  dp = DEGW
  mesh = plsc.VectorSubcoreMesh(core_axis_name="c", subcore_axis_name="s")

  @functools.partial(
      pl.kernel,
      out_type=jax.ShapeDtypeStruct((NC, N, dp), _f32),
      mesh=mesh,
      scratch_types=[
          pltpu.VMEM((2, CH), jnp.int32),
          pltpu.VMEM((2, CH), jnp.int32),
          pltpu.VMEM((2, REM), jnp.int32),
          pltpu.VMEM((CH, dp), _f32),
          pltpu.SemaphoreType.DMA,
          pltpu.SemaphoreType.DMA,
          pltpu.VMEM_SHARED((N, dp), _f32),
      ],
      compiler_params=pltpu.CompilerParams(use_tc_tiling_on_sc=False),
  )
  def k(ew_hbm, ones_hbm, zeros_hbm, out_hbm,
        ib0, ib1, ibr, ones_v, si0, si1, acc_sh):
    cid = lax.axis_index("c")
    sid = lax.axis_index("s")
    wid = sid * NC + cid

    def idx_start(i, ib, sem):
      off = pl.multiple_of(i * CH, CH)
      pltpu.async_copy(ew_hbm.at[wid, :, pl.ds(off, CH)], ib, sem)

    def idx_wait(i, ib, sem):
      off = pl.multiple_of(i * CH, CH)
      pltpu.make_async_copy(ew_hbm.at[wid, :, pl.ds(off, CH)], ib, sem).wait()

    idx_start(0, ib0, si0)
    idx_start(1, ib1, si1)
    pltpu.sync_copy(ones_hbm, ones_v)

    def zbody(k, carry):
      c = sid + k * NS

      @pl.when(c < NRC)
      def _():
        r0 = pl.multiple_of(c * DRC, DRC)
        pltpu.sync_copy(zeros_hbm, acc_sh.at[pl.ds(r0, DRC)])

      return carry

    lax.fori_loop(0, MAXRC, zbody, 0)
    plsc.subcore_barrier()

    def ebody(j, carry):
      i0 = 2 * j
      idx_wait(i0, ib0, si0)
      pltpu.sync_copy(ones_v, acc_sh.at[ib0.at[1]], add=True)
      idx_start(i0 + 2, ib0, si0)
      idx_wait(i0 + 1, ib1, si1)
      pltpu.sync_copy(ones_v, acc_sh.at[ib1.at[1]], add=True)
      idx_start(jnp.minimum(i0 + 3, NFULL - 1), ib1, si1)
      return carry

    lax.fori_loop(0, (NFULL - 2) // 2, ebody, 0)
    idx_wait(NFULL - 2, ib0, si0)
    pltpu.sync_copy(ones_v, acc_sh.at[ib0.at[1]], add=True)
    idx_wait(NFULL - 1, ib1, si1)
    pltpu.sync_copy(ones_v, acc_sh.at[ib1.at[1]], add=True)
    pltpu.sync_copy(ew_hbm.at[wid, :, pl.ds(NFULL * CH, REM)], ibr)
    pltpu.sync_copy(ones_v.at[pl.ds(0, REM)], acc_sh.at[ibr.at[1]], add=True)
    plsc.subcore_barrier()

    def dbody(k, carry):
      c = sid + k * NS

      @pl.when(c < NRC)
      def _():
        r0 = pl.multiple_of(c * DRC, DRC)
        pltpu.sync_copy(acc_sh.at[pl.ds(r0, DRC)], out_hbm.at[cid, pl.ds(r0, DRC)])

      return carry

    lax.fori_loop(0, MAXRC, dbody, 0)

  return k


@functools.lru_cache(maxsize=None)
def _sc_scatter(dp: int):
  mesh = plsc.VectorSubcoreMesh(core_axis_name="c", subcore_axis_name="s")

  @functools.partial(
      pl.kernel,
      out_type=jax.ShapeDtypeStruct((NC, N, dp), _f32),
      mesh=mesh,
      scratch_types=[
          pltpu.VMEM((2, CH), jnp.int32),
          pltpu.VMEM((2, CH), jnp.int32),
          pltpu.VMEM((2, REM), jnp.int32),
          pltpu.VMEM((CH, dp), _f32),
          pltpu.VMEM((CH, dp), _f32),
          pltpu.VMEM((REM, dp), _f32),
          pltpu.SemaphoreType.DMA,
          pltpu.SemaphoreType.DMA,
          pltpu.SemaphoreType.DMA,
          pltpu.SemaphoreType.DMA,
          pltpu.VMEM_SHARED((N, dp), _f32),
      ],
      compiler_params=pltpu.CompilerParams(use_tc_tiling_on_sc=False),
  )
  def k(hs_hbm, ew_hbm, zeros_hbm, out_hbm,
        ib0, ib1, ibr, rows0, rows1, rowsr, si0, si1, sg0, sg1, acc_sh):
    cid = lax.axis_index("c")
    sid = lax.axis_index("s")
    wid = sid * NC + cid

    def idx_start(i, ib, sem):
      off = pl.multiple_of(i * CH, CH)
      pltpu.async_copy(ew_hbm.at[wid, :, pl.ds(off, CH)], ib, sem)

    def idx_wait(i, ib, sem):
      off = pl.multiple_of(i * CH, CH)
      pltpu.make_async_copy(ew_hbm.at[wid, :, pl.ds(off, CH)], ib, sem).wait()

    idx_start(0, ib0, si0)
    idx_start(1, ib1, si1)
    pltpu.sync_copy(zeros_hbm, rows0.at[pl.ds(0, DRC)])

    def zbody(k, carry):
      c = sid + k * NS

      @pl.when(c < NRC)
      def _():
        r0 = pl.multiple_of(c * DRC, DRC)
        pltpu.sync_copy(rows0.at[pl.ds(0, DRC)], acc_sh.at[pl.ds(r0, DRC)])

      return carry

    lax.fori_loop(0, MAXRC, zbody, 0)
    plsc.subcore_barrier()

    idx_wait(0, ib0, si0)
    pltpu.async_copy(hs_hbm.at[ib0.at[0]], rows0, sg0)

    def ebody(j, carry):
      i0 = 2 * j
      pltpu.make_async_copy(hs_hbm.at[ib0.at[0]], rows0, sg0).wait()
      idx_wait(i0 + 1, ib1, si1)
      pltpu.async_copy(hs_hbm.at[ib1.at[0]], rows1, sg1)
      pltpu.sync_copy(rows0, acc_sh.at[ib0.at[1]], add=True)
      idx_start(i0 + 2, ib0, si0)
      pltpu.make_async_copy(hs_hbm.at[ib1.at[0]], rows1, sg1).wait()
      idx_wait(i0 + 2, ib0, si0)
      pltpu.async_copy(hs_hbm.at[ib0.at[0]], rows0, sg0)
      pltpu.sync_copy(rows1, acc_sh.at[ib1.at[1]], add=True)
      idx_start(jnp.minimum(i0 + 3, NFULL - 1), ib1, si1)
      return carry

    lax.fori_loop(0, (NFULL - 2) // 2, ebody, 0)
    pltpu.make_async_copy(hs_hbm.at[ib0.at[0]], rows0, sg0).wait()
    idx_wait(NFULL - 1, ib1, si1)
    pltpu.async_copy(hs_hbm.at[ib1.at[0]], rows1, sg1)
    pltpu.sync_copy(rows0, acc_sh.at[ib0.at[1]], add=True)
    pltpu.sync_copy(ew_hbm.at[wid, :, pl.ds(NFULL * CH, REM)], ibr)
    pltpu.make_async_copy(hs_hbm.at[ib1.at[0]], rows1, sg1).wait()
    pltpu.async_copy(hs_hbm.at[ibr.at[0]], rowsr, sg0)
    pltpu.sync_copy(rows1, acc_sh.at[ib1.at[1]], add=True)
    pltpu.make_async_copy(hs_hbm.at[ibr.at[0]], rowsr, sg0).wait()
    pltpu.sync_copy(rowsr, acc_sh.at[ibr.at[1]], add=True)
    plsc.subcore_barrier()

    def dbody(k, carry):
      c = sid + k * NS

      @pl.when(c < NRC)
      def _():
        r0 = pl.multiple_of(c * DRC, DRC)
        pltpu.sync_copy(acc_sh.at[pl.ds(r0, DRC)], out_hbm.at[cid, pl.ds(r0, DRC)])

      return carry

    lax.fori_loop(0, MAXRC, dbody, 0)

  return k


def _prep_body(x_ref, w_ref, degp_ref, ts_ref, dinv_ref):
  deg = degp_ref[0][:, 0:1] + degp_ref[1][:, 0:1] + 1.0
  dinv = deg ** -0.5
  dinv_ref[...] = dinv
  t = jnp.dot(x_ref[...], w_ref[...], preferred_element_type=_f32)
  ts_ref[...] = t * dinv


def _prep_call(x, w0p, degp):
  return pl.pallas_call(
      _prep_body,
      grid=(NB,),
      in_specs=[
          pl.BlockSpec((BN, 128), lambda i: (i, 0)),
          pl.BlockSpec((128, 112), lambda i: (0, 0)),
          pl.BlockSpec((2, BN, 16), lambda i: (0, i, 0)),
      ],
      out_specs=[
          pl.BlockSpec((BN, 112), lambda i: (i, 0)),
          pl.BlockSpec((BN, 1), lambda i: (i, 0)),
      ],
      out_shape=[
          jax.ShapeDtypeStruct((N, 112), _f32),
          jax.ShapeDtypeStruct((N, 1), _f32),
      ],
  )(x, w0p, degp)


def _comb_body(acc_ref, ts_ref, dinv_ref, b_ref, out_ref):
  dinv = dinv_ref[...]
  h = jnp.maximum((acc_ref[0] + acc_ref[1] + ts_ref[...]) * dinv + b_ref[...],
                  0.0)
  out_ref[...] = h * dinv


def _comb_call(acc, ts, dinv, bp):
  d = ts.shape[1]
  return pl.pallas_call(
      _comb_body,
      grid=(NB,),
      in_specs=[
          pl.BlockSpec((2, BN, d), lambda i: (0, i, 0)),
          pl.BlockSpec((BN, d), lambda i: (i, 0)),
          pl.BlockSpec((BN, 1), lambda i: (i, 0)),
          pl.BlockSpec((1, d), lambda i: (0, 0)),
      ],
      out_specs=pl.BlockSpec((BN, d), lambda i: (i, 0)),
      out_shape=jax.ShapeDtypeStruct((N, d), _f32),
  )(acc, ts, dinv, bp)


def _mm2_body(acc_ref, ts_ref, dinv_ref, w1_ref, b1_ref, w2_ref, out_ref):
  dinv = dinv_ref[...]
  p = (acc_ref[0] + acc_ref[1] + ts_ref[...]) * dinv
  h = jnp.maximum(
      jnp.dot(p, w1_ref[...], preferred_element_type=_f32) + b1_ref[...], 0.0)
  t = jnp.dot(h, w2_ref[...], preferred_element_type=_f32)
  out_ref[...] = t * dinv


def _mm2_call(acc, ts, dinv, w1p, b1, w2p):
  din = ts.shape[1]
  dmid = w1p.shape[1]
  dout = w2p.shape[1]
  return pl.pallas_call(
      _mm2_body,
      grid=(NB,),
      in_specs=[
          pl.BlockSpec((2, BN, din), lambda i: (0, i, 0)),
          pl.BlockSpec((BN, din), lambda i: (i, 0)),
          pl.BlockSpec((BN, 1), lambda i: (i, 0)),
          pl.BlockSpec((din, dmid), lambda i: (0, 0)),
          pl.BlockSpec((1, dmid), lambda i: (0, 0)),
          pl.BlockSpec((dmid, dout), lambda i: (0, 0)),
      ],
      out_specs=pl.BlockSpec((BN, dout), lambda i: (i, 0)),
      out_shape=jax.ShapeDtypeStruct((N, dout), _f32),
  )(acc, ts, dinv, w1p, b1, w2p)


def _mm1_body(acc_ref, ts_ref, dinv_ref, b_ref, w_ref, out_ref):
  dinv = dinv_ref[...]
  h = jnp.maximum((acc_ref[0] + acc_ref[1] + ts_ref[...]) * dinv + b_ref[...],
                  0.0)
  t = jnp.dot(h, w_ref[...], preferred_element_type=_f32)
  out_ref[...] = t * dinv


def _mm1_call(acc, ts, dinv, bp, wp):
  din = ts.shape[1]
  dout = wp.shape[1]
  return pl.pallas_call(
      _mm1_body,
      grid=(NB,),
      in_specs=[
          pl.BlockSpec((2, BN, din), lambda i: (0, i, 0)),
          pl.BlockSpec((BN, din), lambda i: (i, 0)),
          pl.BlockSpec((BN, 1), lambda i: (i, 0)),
          pl.BlockSpec((1, din), lambda i: (0, 0)),
          pl.BlockSpec((din, dout), lambda i: (0, 0)),
      ],
      out_specs=pl.BlockSpec((BN, dout), lambda i: (i, 0)),
      out_shape=jax.ShapeDtypeStruct((N, dout), _f32),
  )(acc, ts, dinv, bp, wp)


def _final_body(acc_ref, ts_ref, dinv_ref, bm_ref, b3_ref,
                wd0_ref, bd0_ref, wd1_ref, bd1_ref, wd2_ref, bd2_ref,
                wd3_ref, bd3_ref, wn_ref, bnn_ref,
                t_ref, mfa_ref, mfw_ref,
                outa_ref, outw_ref, g_ref):
  i = pl.program_id(0)
  dinv = dinv_ref[...]
  h4 = jnp.maximum((acc_ref[0] + acc_ref[1] + ts_ref[...]) * dinv + b3_ref[...],
                   0.0)
  onehot = (bm_ref[...] == lax.broadcasted_iota(jnp.int32, (BN, B), 1)
            ).astype(_f32)
  contrib = lax.dot_general(onehot, h4, (((0,), (0,)), ((), ())),
                            preferred_element_type=_f32)

  @pl.when(i == 0)
  def _():
    g_ref[...] = contrib

  @pl.when(i > 0)
  def _():
    g_ref[...] = g_ref[...] + contrib

  @pl.when(i == pl.num_programs(0) - 1)
  def _():
    g = jnp.maximum(g_ref[...], 0.0)
    g = jnp.maximum(
        jnp.dot(g, wd0_ref[...], preferred_element_type=_f32) + bd0_ref[...],
        0.0)
    g = jnp.maximum(
        jnp.dot(g, wd1_ref[...], preferred_element_type=_f32) + bd1_ref[...],
        0.0)
    g = jnp.maximum(
        jnp.dot(g, wd2_ref[...], preferred_element_type=_f32) + bd2_ref[...],
        0.0)
    g = jnp.maximum(
        jnp.dot(g, wd3_ref[...], preferred_element_type=_f32) + bd3_ref[...],
        0.0)
    coef = jnp.dot(g, wn_ref[...], preferred_element_type=_f32) + bnn_ref[...]
    alpha = coef[:, 0:1]
    b12 = coef[:, 1:2]
    b21 = coef[:, 2:3]
    sig = 1.0 / (1.0 + jnp.exp(-alpha))
    alpha = 0.2 * (1.0 + sig / 10.0 * (0.47 / 0.2))
    r_t = 62.36367 * (t_ref[...] + 273.15)
    tau12 = b12 / r_t
    tau21 = b21 / r_t
    g12 = jnp.exp(-tau12 * alpha)
    g21 = jnp.exp(-tau21 * alpha)
    mfa = mfa_ref[...]
    mfw = mfw_ref[...]
    den12 = mfw + mfa * g12
    den21 = mfa + mfw * g21
    q12 = g12 / den12
    q21 = g21 / den21
    outa_ref[...] = mfw * mfw * (tau21 * q21 * q21 + tau12 * g12 / (den12 * den12))
    outw_ref[...] = mfa * mfa * (tau12 * q12 * q12 + tau21 * g21 / (den21 * den21))


def _final_call(acc, ts, dinv, bm2, b3p, wd0p, bd0, wd1, bd1, wd2, bd2,
                wd3, bd3, wn, bnn, temp, mfa, mfw):
  full = lambda shape: pl.BlockSpec(shape, lambda i: tuple(0 for _ in shape))
  return pl.pallas_call(
      _final_body,
      grid=(NB,),
      in_specs=[
          pl.BlockSpec((2, BN, 144), lambda i: (0, i, 0)),
          pl.BlockSpec((BN, 144), lambda i: (i, 0)),
          pl.BlockSpec((BN, 1), lambda i: (i, 0)),
          pl.BlockSpec((BN, 1), lambda i: (i, 0)),
          full((1, 144)),
          full((144, 260)), full((1, 260)),
          full((260, 60)), full((1, 60)),
          full((60, 180)), full((1, 180)),
          full((180, 100)), full((1, 100)),
          full((100, 3)), full((1, 3)),
          full((B, 1)), full((B, 1)), full((B, 1)),
      ],
      out_specs=[
          pl.BlockSpec((B, 1), lambda i: (0, 0)),
          pl.BlockSpec((B, 1), lambda i: (0, 0)),
      ],
      out_shape=[
          jax.ShapeDtypeStruct((B, 1), _f32),
          jax.ShapeDtypeStruct((B, 1), _f32),
      ],
      scratch_shapes=[pltpu.VMEM((B, 144), _f32)],
  )(acc, ts, dinv, bm2, b3p, wd0p, bd0, wd1, bd1, wd2, bd2, wd3, bd3,
    wn, bnn, temp, mfa, mfw)


def kernel(x, edge_indices, batch_mapping, temperature, mean, std,
           mole_frac_amine, mole_frac_water,
           Wg0, bg0, Wg1, bg1, Wg2, bg2, Wg3, bg3,
           Wd0, bd0, Wd1, bd1, Wd2, bd2, Wd3, bd3, Wn, bn):
  ew = edge_indices.reshape(2, NW, EPW).transpose(1, 0, 2)

  ones16 = jnp.ones((CH, DEGW), _f32)
  z16 = jnp.zeros((DRC, DEGW), _f32)
  z112 = jnp.zeros((DRC, 112), _f32)
  z144 = jnp.zeros((DRC, 144), _f32)

  degp = _sc_degree()(ew, ones16, z16)

  w0p = jnp.pad(Wg0, ((0, 0), (0, 12)))
  ts0, dinv = _prep_call(x, w0p, degp)
  acc0 = _sc_scatter(112)(ts0, ew, z112)

  b0p = jnp.pad(bg0, (0, 12)).reshape(1, 112)
  ts1 = _comb_call(acc0, ts0, dinv, b0p)
  acc1 = _sc_scatter(112)(ts1, ew, z112)

  w1p = jnp.pad(Wg1, ((0, 12), (0, 0)))
  w2p = jnp.pad(Wg2, ((0, 0), (0, 4)))
  ts2 = _mm2_call(acc1, ts1, dinv, w1p, bg1.reshape(1, 420), w2p)
  acc2 = _sc_scatter(144)(ts2, ew, z144)

  b2p = jnp.pad(bg2, (0, 4)).reshape(1, 144)
  w3p = jnp.pad(Wg3, ((0, 4), (0, 4)))
  ts3 = _mm1_call(acc2, ts2, dinv, b2p, w3p)
  acc3 = _sc_scatter(144)(ts3, ew, z144)

  b3p = jnp.pad(bg3, (0, 4)).reshape(1, 144)
  wd0p = jnp.pad(Wd0, ((0, 4), (0, 0)))
  ln_a, ln_w = _final_call(
      acc3, ts3, dinv, batch_mapping.reshape(N, 1).astype(jnp.int32), b3p,
      wd0p, bd0.reshape(1, -1), Wd1, bd1.reshape(1, -1),
      Wd2, bd2.reshape(1, -1), Wd3, bd3.reshape(1, -1),
      Wn, bn.reshape(1, -1), temperature.reshape(B, 1),
      mole_frac_amine.reshape(B, 1), mole_frac_water.reshape(B, 1))
  return ln_a.reshape(B), ln_w.reshape(B)
